# Initial kernel scaffold; baseline (speedup 1.0000x reference)
import jax
import jax.numpy as jnp
from jax.experimental import pallas as pl


def kernel(images, features_0, W_conv, b_conv, W_cls, b_cls, W_reg, b_reg):
    raise NotImplementedError("write your pallas kernel here")


# speedup vs baseline: 5.3034x; 5.3034x over previous
import numpy as np
import jax
import jax.numpy as jnp
from jax.experimental import pallas as pl
from jax.experimental.pallas import tpu as pltpu

H = 200
W = 200
A = 3
C = 64
IMG = 800.0
STRIDE = 4.0
PRE_NMS = 2000
POST_NMS = 1000
NMS_THRESH = 0.7
MIN_SIZE = 1e-3
BBOX_XFORM_CLIP = float(np.log(1000.0 / 16.0))
HW = H * W
HW_PAD = 40960
BN = 4096
NP = 2048


def _anchors_np():
    scales = np.array([32.0], np.float32)
    ratios = np.array([0.5, 1.0, 2.0], np.float32)
    h_r = np.sqrt(ratios)
    w_r = 1.0 / h_r
    ws = (w_r[:, None] * scales[None, :]).reshape(-1)
    hs = (h_r[:, None] * scales[None, :]).reshape(-1)
    base = np.round(np.stack([-ws, -hs, ws, hs], 1) / 2.0)
    sx = np.arange(W, dtype=np.float32) * STRIDE
    sy = np.arange(H, dtype=np.float32) * STRIDE
    yy, xx = np.meshgrid(sy, sx, indexing="ij")
    shifts = np.stack([xx.reshape(-1), yy.reshape(-1)] * 2, 1)
    return (shifts[:, None, :] + base[None, :, :]).reshape(-1, 4)


_ANCHORS = _anchors_np()


def _head_body(x_ref, wc_ref, bc_ref, wcls_ref, bcls_ref, wreg_ref, breg_ref,
               obj_ref, reg_ref):
    x = x_ref[...]
    t = jnp.dot(wc_ref[...], x, preferred_element_type=jnp.float32) + bc_ref[...]
    t = jnp.maximum(t, 0.0)
    obj_ref[...] = (jnp.dot(wcls_ref[...], t, preferred_element_type=jnp.float32)
                    + bcls_ref[...])
    reg_ref[...] = (jnp.dot(wreg_ref[...], t, preferred_element_type=jnp.float32)
                    + breg_ref[...])


def _decode_body(lg_ref, d_ref, a_ref, box_ref, sc_ref):
    a0 = a_ref[0:1, :]
    a1 = a_ref[1:2, :]
    a2 = a_ref[2:3, :]
    a3 = a_ref[3:4, :]
    d0 = d_ref[0:1, :]
    d1 = d_ref[1:2, :]
    d2 = d_ref[2:3, :]
    d3 = d_ref[3:4, :]
    wdt = a2 - a0
    hgt = a3 - a1
    cx = a0 + 0.5 * wdt
    cy = a1 + 0.5 * hgt
    dw = jnp.minimum(d2, BBOX_XFORM_CLIP)
    dh = jnp.minimum(d3, BBOX_XFORM_CLIP)
    pcx = d0 * wdt + cx
    pcy = d1 * hgt + cy
    pw = jnp.exp(dw) * wdt
    ph = jnp.exp(dh) * hgt
    x1 = jnp.clip(pcx - 0.5 * pw, 0.0, IMG)
    y1 = jnp.clip(pcy - 0.5 * ph, 0.0, IMG)
    x2 = jnp.clip(pcx + 0.5 * pw, 0.0, IMG)
    y2 = jnp.clip(pcy + 0.5 * ph, 0.0, IMG)
    box_ref[0:1, :] = x1
    box_ref[1:2, :] = y1
    box_ref[2:3, :] = x2
    box_ref[3:4, :] = y2
    valid = ((x2 - x1) >= MIN_SIZE) & ((y2 - y1) >= MIN_SIZE)
    sc_ref[...] = jnp.where(valid, jax.nn.sigmoid(lg_ref[...]), -jnp.inf)


def _nms_body(bx_ref, bc_ref, sc_ref, out_ref, iou_ref):
    x1 = bx_ref[0:1, :]
    y1 = bx_ref[1:2, :]
    x2 = bx_ref[2:3, :]
    y2 = bx_ref[3:4, :]
    area = (x2 - x1) * (y2 - y1)
    RB = 128

    def fill(b, _):
        r0 = b * RB
        rx1 = bc_ref[pl.ds(r0, RB), 0:1]
        ry1 = bc_ref[pl.ds(r0, RB), 1:2]
        rx2 = bc_ref[pl.ds(r0, RB), 2:3]
        ry2 = bc_ref[pl.ds(r0, RB), 3:4]
        iw = jnp.maximum(jnp.minimum(rx2, x2) - jnp.maximum(rx1, x1), 0.0)
        ih = jnp.maximum(jnp.minimum(ry2, y2) - jnp.maximum(ry1, y1), 0.0)
        inter = iw * ih
        r_area = (rx2 - rx1) * (ry2 - ry1)
        iou_ref[pl.ds(r0, RB), :] = inter / (r_area + area - inter + 1e-9)
        return 0

    jax.lax.fori_loop(0, NP // RB, fill, 0)

    scores = sc_ref[...]
    ar = jax.lax.broadcasted_iota(jnp.int32, (1, NP), 1)
    ninf = jnp.float32(-jnp.inf)

    def body(i, s):
        row = iou_ref[pl.ds(i, 1), :]
        keep_i = jnp.max(jnp.where(ar == i, s, ninf)) > ninf
        sup = (row > NMS_THRESH) & (ar > i) & keep_i
        return jnp.where(sup, ninf, s)

    out_ref[...] = jax.lax.fori_loop(0, NP, body, scores)


def kernel(images, features_0, W_conv, b_conv, W_cls, b_cls, W_reg, b_reg):
    xp = jnp.pad(features_0[0], ((0, 0), (1, 1), (1, 1)))
    cols = jnp.stack([xp[:, ky:ky + H, kx:kx + W]
                      for ky in range(3) for kx in range(3)], 0)
    X = jnp.pad(cols.reshape(9 * C, HW), ((0, 0), (0, HW_PAD - HW)))
    W2 = W_conv.transpose(0, 2, 3, 1).reshape(C, 9 * C)

    obj, reg = pl.pallas_call(
        _head_body,
        grid=(HW_PAD // BN,),
        in_specs=[
            pl.BlockSpec((9 * C, BN), lambda j: (0, j)),
            pl.BlockSpec((C, 9 * C), lambda j: (0, 0)),
            pl.BlockSpec((C, 1), lambda j: (0, 0)),
            pl.BlockSpec((A, C), lambda j: (0, 0)),
            pl.BlockSpec((A, 1), lambda j: (0, 0)),
            pl.BlockSpec((A * 4, C), lambda j: (0, 0)),
            pl.BlockSpec((A * 4, 1), lambda j: (0, 0)),
        ],
        out_specs=[
            pl.BlockSpec((A, BN), lambda j: (0, j)),
            pl.BlockSpec((A * 4, BN), lambda j: (0, j)),
        ],
        out_shape=[
            jax.ShapeDtypeStruct((A, HW_PAD), jnp.float32),
            jax.ShapeDtypeStruct((A * 4, HW_PAD), jnp.float32),
        ],
    )(X, W2, b_conv[:, None], W_cls, b_cls[:, None], W_reg, b_reg[:, None])

    obj_flat = obj[:, :HW].T.reshape(-1)
    deltas_flat = reg[:, :HW].T.reshape(HW * A, 4)

    top_logits, idx = jax.lax.top_k(obj_flat, PRE_NMS)
    anchors = jnp.asarray(_ANCHORS)
    a_sel = jnp.take(anchors, idx, axis=0).T
    d_sel = jnp.take(deltas_flat, idx, axis=0).T

    boxes_t, scores_m = pl.pallas_call(
        _decode_body,
        out_shape=[
            jax.ShapeDtypeStruct((4, PRE_NMS), jnp.float32),
            jax.ShapeDtypeStruct((1, PRE_NMS), jnp.float32),
        ],
    )(top_logits[None, :], d_sel, a_sel, )

    scores_m = scores_m[0]
    order = jnp.argsort(-scores_m)
    boxes_o = jnp.take(boxes_t.T, order, axis=0)
    scores_o = jnp.take(scores_m, order)

    bx = jnp.pad(boxes_o.T, ((0, 0), (0, NP - PRE_NMS)))
    bc = jnp.pad(boxes_o, ((0, NP - PRE_NMS), (0, 0)))
    sc = jnp.pad(scores_o, (0, NP - PRE_NMS), constant_values=-jnp.inf)[None, :]

    masked = pl.pallas_call(
        _nms_body,
        out_shape=jax.ShapeDtypeStruct((1, NP), jnp.float32),
        scratch_shapes=[pltpu.VMEM((NP, NP), jnp.float32)],
    )(bx, bc, sc)

    final_scores, fidx = jax.lax.top_k(masked[0, :PRE_NMS], POST_NMS)
    final_boxes = jnp.take(boxes_o, fidx, axis=0)
    return final_boxes, final_scores, obj_flat, deltas_flat


# speedup vs baseline: 5.6285x; 1.0613x over previous
import numpy as np
import jax
import jax.numpy as jnp
from jax.experimental import pallas as pl
from jax.experimental.pallas import tpu as pltpu

H = 200
W = 200
A = 3
C = 64
IMG = 800.0
STRIDE = 4.0
PRE_NMS = 2000
POST_NMS = 1000
NMS_THRESH = 0.7
MIN_SIZE = 1e-3
BBOX_XFORM_CLIP = float(np.log(1000.0 / 16.0))
HW = H * W
HW_PAD = 40960
BN = 4096
NP = 2048


def _anchors_np():
    scales = np.array([32.0], np.float32)
    ratios = np.array([0.5, 1.0, 2.0], np.float32)
    h_r = np.sqrt(ratios)
    w_r = 1.0 / h_r
    ws = (w_r[:, None] * scales[None, :]).reshape(-1)
    hs = (h_r[:, None] * scales[None, :]).reshape(-1)
    base = np.round(np.stack([-ws, -hs, ws, hs], 1) / 2.0)
    sx = np.arange(W, dtype=np.float32) * STRIDE
    sy = np.arange(H, dtype=np.float32) * STRIDE
    yy, xx = np.meshgrid(sy, sx, indexing="ij")
    shifts = np.stack([xx.reshape(-1), yy.reshape(-1)] * 2, 1)
    return (shifts[:, None, :] + base[None, :, :]).reshape(-1, 4)


_ANCHORS = _anchors_np()


def _head_body(x_ref, wc_ref, bc_ref, wcls_ref, bcls_ref, wreg_ref, breg_ref,
               obj_ref, reg_ref):
    x = x_ref[...]
    t = jnp.dot(wc_ref[...], x, preferred_element_type=jnp.float32) + bc_ref[...]
    t = jnp.maximum(t, 0.0)
    obj_ref[...] = (jnp.dot(wcls_ref[...], t, preferred_element_type=jnp.float32)
                    + bcls_ref[...])
    reg_ref[...] = (jnp.dot(wreg_ref[...], t, preferred_element_type=jnp.float32)
                    + breg_ref[...])


def _decode_body(lg_ref, d_ref, a_ref, box_ref, sc_ref):
    a0 = a_ref[0:1, :]
    a1 = a_ref[1:2, :]
    a2 = a_ref[2:3, :]
    a3 = a_ref[3:4, :]
    d0 = d_ref[0:1, :]
    d1 = d_ref[1:2, :]
    d2 = d_ref[2:3, :]
    d3 = d_ref[3:4, :]
    wdt = a2 - a0
    hgt = a3 - a1
    cx = a0 + 0.5 * wdt
    cy = a1 + 0.5 * hgt
    dw = jnp.minimum(d2, BBOX_XFORM_CLIP)
    dh = jnp.minimum(d3, BBOX_XFORM_CLIP)
    pcx = d0 * wdt + cx
    pcy = d1 * hgt + cy
    pw = jnp.exp(dw) * wdt
    ph = jnp.exp(dh) * hgt
    x1 = jnp.clip(pcx - 0.5 * pw, 0.0, IMG)
    y1 = jnp.clip(pcy - 0.5 * ph, 0.0, IMG)
    x2 = jnp.clip(pcx + 0.5 * pw, 0.0, IMG)
    y2 = jnp.clip(pcy + 0.5 * ph, 0.0, IMG)
    box_ref[0:1, :] = x1
    box_ref[1:2, :] = y1
    box_ref[2:3, :] = x2
    box_ref[3:4, :] = y2
    valid = ((x2 - x1) >= MIN_SIZE) & ((y2 - y1) >= MIN_SIZE)
    sc_ref[...] = jnp.where(valid, jax.nn.sigmoid(lg_ref[...]), -jnp.inf)


def _nms_body(bx_ref, bc_ref, sc_ref, out_ref, tin_ref):
    x1 = bx_ref[0:1, :]
    y1 = bx_ref[1:2, :]
    x2 = bx_ref[2:3, :]
    y2 = bx_ref[3:4, :]
    area = (x2 - x1) * (y2 - y1)
    RB = 128
    ar = jax.lax.broadcasted_iota(jnp.int32, (1, NP), 1)
    arB = jax.lax.broadcasted_iota(jnp.int32, (1, RB), 1)
    ninf = jnp.float32(-jnp.inf)
    out_ref[...] = sc_ref[...]

    def block(b, _):
        r0 = b * RB
        rx1 = bc_ref[pl.ds(r0, RB), 0:1]
        ry1 = bc_ref[pl.ds(r0, RB), 1:2]
        rx2 = bc_ref[pl.ds(r0, RB), 2:3]
        ry2 = bc_ref[pl.ds(r0, RB), 3:4]
        iw = jnp.maximum(jnp.minimum(rx2, x2) - jnp.maximum(rx1, x1), 0.0)
        ih = jnp.maximum(jnp.minimum(ry2, y2) - jnp.maximum(ry1, y1), 0.0)
        inter = iw * ih
        r_area = (rx2 - rx1) * (ry2 - ry1)
        tile = inter / (r_area + area - inter + 1e-9)
        cx1 = bx_ref[0:1, pl.ds(r0, RB)]
        cy1 = bx_ref[1:2, pl.ds(r0, RB)]
        cx2 = bx_ref[2:3, pl.ds(r0, RB)]
        cy2 = bx_ref[3:4, pl.ds(r0, RB)]
        tiw = jnp.maximum(jnp.minimum(rx2, cx2) - jnp.maximum(rx1, cx1), 0.0)
        tih = jnp.maximum(jnp.minimum(ry2, cy2) - jnp.maximum(ry1, cy1), 0.0)
        tinter = tiw * tih
        carea = (cx2 - cx1) * (cy2 - cy1)
        tin_ref[...] = tinter / (r_area + carea - tinter + 1e-9)
        sb0 = out_ref[0:1, pl.ds(r0, RB)]

        def inner(i, sb):
            row = tin_ref[pl.ds(i, 1), :]
            keep_i = jnp.max(jnp.where(arB == i, sb, ninf)) > ninf
            sup = (row > NMS_THRESH) & (arB > i) & keep_i
            return jnp.where(sup, ninf, sb)

        sb = jax.lax.fori_loop(0, RB, inner, sb0)
        out_ref[0:1, pl.ds(r0, RB)] = sb
        keptf = jnp.isfinite(sb).astype(jnp.float32)
        gt = (tile > NMS_THRESH).astype(jnp.float32)
        cnt = jnp.dot(keptf, gt, preferred_element_type=jnp.float32)
        s = out_ref[...]
        out_ref[...] = jnp.where((cnt > 0.0) & (ar >= r0 + RB), ninf, s)
        return 0

    jax.lax.fori_loop(0, NP // RB, block, 0)


def kernel(images, features_0, W_conv, b_conv, W_cls, b_cls, W_reg, b_reg):
    xp = jnp.pad(features_0[0], ((0, 0), (1, 1), (1, 1)))
    cols = jnp.stack([xp[:, ky:ky + H, kx:kx + W]
                      for ky in range(3) for kx in range(3)], 0)
    X = jnp.pad(cols.reshape(9 * C, HW), ((0, 0), (0, HW_PAD - HW)))
    W2 = W_conv.transpose(0, 2, 3, 1).reshape(C, 9 * C)

    obj, reg = pl.pallas_call(
        _head_body,
        grid=(HW_PAD // BN,),
        in_specs=[
            pl.BlockSpec((9 * C, BN), lambda j: (0, j)),
            pl.BlockSpec((C, 9 * C), lambda j: (0, 0)),
            pl.BlockSpec((C, 1), lambda j: (0, 0)),
            pl.BlockSpec((A, C), lambda j: (0, 0)),
            pl.BlockSpec((A, 1), lambda j: (0, 0)),
            pl.BlockSpec((A * 4, C), lambda j: (0, 0)),
            pl.BlockSpec((A * 4, 1), lambda j: (0, 0)),
        ],
        out_specs=[
            pl.BlockSpec((A, BN), lambda j: (0, j)),
            pl.BlockSpec((A * 4, BN), lambda j: (0, j)),
        ],
        out_shape=[
            jax.ShapeDtypeStruct((A, HW_PAD), jnp.float32),
            jax.ShapeDtypeStruct((A * 4, HW_PAD), jnp.float32),
        ],
    )(X, W2, b_conv[:, None], W_cls, b_cls[:, None], W_reg, b_reg[:, None])

    obj_flat = obj[:, :HW].T.reshape(-1)
    deltas_flat = reg[:, :HW].T.reshape(HW * A, 4)

    top_logits, idx = jax.lax.top_k(obj_flat, PRE_NMS)
    anchors = jnp.asarray(_ANCHORS)
    a_sel = jnp.take(anchors, idx, axis=0).T
    d_sel = jnp.take(deltas_flat, idx, axis=0).T

    boxes_t, scores_m = pl.pallas_call(
        _decode_body,
        out_shape=[
            jax.ShapeDtypeStruct((4, PRE_NMS), jnp.float32),
            jax.ShapeDtypeStruct((1, PRE_NMS), jnp.float32),
        ],
    )(top_logits[None, :], d_sel, a_sel, )

    scores_m = scores_m[0]
    order = jnp.argsort(-scores_m)
    boxes_o = jnp.take(boxes_t.T, order, axis=0)
    scores_o = jnp.take(scores_m, order)

    bx = jnp.pad(boxes_o.T, ((0, 0), (0, NP - PRE_NMS)))
    bc = jnp.pad(boxes_o, ((0, NP - PRE_NMS), (0, 0)))
    sc = jnp.pad(scores_o, (0, NP - PRE_NMS), constant_values=-jnp.inf)[None, :]

    masked = pl.pallas_call(
        _nms_body,
        out_shape=jax.ShapeDtypeStruct((1, NP), jnp.float32),
        scratch_shapes=[pltpu.VMEM((128, 128), jnp.float32)],
    )(bx, bc, sc)

    final_scores, fidx = jax.lax.top_k(masked[0, :PRE_NMS], POST_NMS)
    final_boxes = jnp.take(boxes_o, fidx, axis=0)
    return final_boxes, final_scores, obj_flat, deltas_flat


# speedup vs baseline: 5.7817x; 1.0272x over previous
import numpy as np
import jax
import jax.numpy as jnp
from jax.experimental import pallas as pl
from jax.experimental.pallas import tpu as pltpu

H = 200
W = 200
A = 3
C = 64
IMG = 800.0
STRIDE = 4.0
PRE_NMS = 2000
POST_NMS = 1000
NMS_THRESH = 0.7
MIN_SIZE = 1e-3
BBOX_XFORM_CLIP = float(np.log(1000.0 / 16.0))
HW = H * W
WP = 208
HW_PAD = 200 * 208
BN = 3200
NP = 2048


def _anchors_np():
    scales = np.array([32.0], np.float32)
    ratios = np.array([0.5, 1.0, 2.0], np.float32)
    h_r = np.sqrt(ratios)
    w_r = 1.0 / h_r
    ws = (w_r[:, None] * scales[None, :]).reshape(-1)
    hs = (h_r[:, None] * scales[None, :]).reshape(-1)
    base = np.round(np.stack([-ws, -hs, ws, hs], 1) / 2.0)
    sx = np.arange(W, dtype=np.float32) * STRIDE
    sy = np.arange(H, dtype=np.float32) * STRIDE
    yy, xx = np.meshgrid(sy, sx, indexing="ij")
    shifts = np.stack([xx.reshape(-1), yy.reshape(-1)] * 2, 1)
    return (shifts[:, None, :] + base[None, :, :]).reshape(-1, 4)


_ANCHORS = _anchors_np()


def _head_body(x_ref, wc_ref, bc_ref, wcls_ref, bcls_ref, wreg_ref, breg_ref,
               obj_ref, reg_ref):
    x = x_ref[...]
    t = jnp.dot(wc_ref[...], x, preferred_element_type=jnp.float32) + bc_ref[...]
    t = jnp.maximum(t, 0.0)
    obj_ref[...] = (jnp.dot(wcls_ref[...], t, preferred_element_type=jnp.float32)
                    + bcls_ref[...])
    reg_ref[...] = (jnp.dot(wreg_ref[...], t, preferred_element_type=jnp.float32)
                    + breg_ref[...])


def _decode_body(lg_ref, d_ref, a_ref, box_ref, sc_ref):
    a0 = a_ref[0:1, :]
    a1 = a_ref[1:2, :]
    a2 = a_ref[2:3, :]
    a3 = a_ref[3:4, :]
    d0 = d_ref[0:1, :]
    d1 = d_ref[1:2, :]
    d2 = d_ref[2:3, :]
    d3 = d_ref[3:4, :]
    wdt = a2 - a0
    hgt = a3 - a1
    cx = a0 + 0.5 * wdt
    cy = a1 + 0.5 * hgt
    dw = jnp.minimum(d2, BBOX_XFORM_CLIP)
    dh = jnp.minimum(d3, BBOX_XFORM_CLIP)
    pcx = d0 * wdt + cx
    pcy = d1 * hgt + cy
    pw = jnp.exp(dw) * wdt
    ph = jnp.exp(dh) * hgt
    x1 = jnp.clip(pcx - 0.5 * pw, 0.0, IMG)
    y1 = jnp.clip(pcy - 0.5 * ph, 0.0, IMG)
    x2 = jnp.clip(pcx + 0.5 * pw, 0.0, IMG)
    y2 = jnp.clip(pcy + 0.5 * ph, 0.0, IMG)
    box_ref[0:1, :] = x1
    box_ref[1:2, :] = y1
    box_ref[2:3, :] = x2
    box_ref[3:4, :] = y2
    valid = ((x2 - x1) >= MIN_SIZE) & ((y2 - y1) >= MIN_SIZE)
    sc_ref[...] = jnp.where(valid, jax.nn.sigmoid(lg_ref[...]), -jnp.inf)


def _nms_body(bx_ref, bc_ref, sc_ref, out_ref, tin_ref):
    x1 = bx_ref[0:1, :]
    y1 = bx_ref[1:2, :]
    x2 = bx_ref[2:3, :]
    y2 = bx_ref[3:4, :]
    area = (x2 - x1) * (y2 - y1)
    RB = 128
    ar = jax.lax.broadcasted_iota(jnp.int32, (1, NP), 1)
    arB = jax.lax.broadcasted_iota(jnp.int32, (1, RB), 1)
    ninf = jnp.float32(-jnp.inf)
    out_ref[...] = sc_ref[...]

    def block(b, _):
        r0 = b * RB
        rx1 = bc_ref[pl.ds(r0, RB), 0:1]
        ry1 = bc_ref[pl.ds(r0, RB), 1:2]
        rx2 = bc_ref[pl.ds(r0, RB), 2:3]
        ry2 = bc_ref[pl.ds(r0, RB), 3:4]
        iw = jnp.maximum(jnp.minimum(rx2, x2) - jnp.maximum(rx1, x1), 0.0)
        ih = jnp.maximum(jnp.minimum(ry2, y2) - jnp.maximum(ry1, y1), 0.0)
        inter = iw * ih
        r_area = (rx2 - rx1) * (ry2 - ry1)
        tile = inter / (r_area + area - inter + 1e-9)
        cx1 = bx_ref[0:1, pl.ds(r0, RB)]
        cy1 = bx_ref[1:2, pl.ds(r0, RB)]
        cx2 = bx_ref[2:3, pl.ds(r0, RB)]
        cy2 = bx_ref[3:4, pl.ds(r0, RB)]
        tiw = jnp.maximum(jnp.minimum(rx2, cx2) - jnp.maximum(rx1, cx1), 0.0)
        tih = jnp.maximum(jnp.minimum(ry2, cy2) - jnp.maximum(ry1, cy1), 0.0)
        tinter = tiw * tih
        carea = (cx2 - cx1) * (cy2 - cy1)
        tin_ref[...] = tinter / (r_area + carea - tinter + 1e-9)
        sb0 = out_ref[0:1, pl.ds(r0, RB)]

        def inner(i, sb):
            row = tin_ref[pl.ds(i, 1), :]
            keep_i = jnp.max(jnp.where(arB == i, sb, ninf)) > ninf
            sup = (row > NMS_THRESH) & (arB > i) & keep_i
            return jnp.where(sup, ninf, sb)

        sb = jax.lax.fori_loop(0, RB, inner, sb0)
        out_ref[0:1, pl.ds(r0, RB)] = sb
        keptf = jnp.isfinite(sb).astype(jnp.float32)
        gt = (tile > NMS_THRESH).astype(jnp.float32)
        cnt = jnp.dot(keptf, gt, preferred_element_type=jnp.float32)
        s = out_ref[...]
        out_ref[...] = jnp.where((cnt > 0.0) & (ar >= r0 + RB), ninf, s)
        return 0

    jax.lax.fori_loop(0, NP // RB, block, 0)


def kernel(images, features_0, W_conv, b_conv, W_cls, b_cls, W_reg, b_reg):
    xp = jnp.pad(features_0[0], ((0, 0), (1, 1), (1, 9)))
    cols = jnp.stack([xp[:, ky:ky + H, kx:kx + WP]
                      for ky in range(3) for kx in range(3)], 0)
    X = cols.reshape(9 * C, HW_PAD)
    W2 = W_conv.transpose(0, 2, 3, 1).reshape(C, 9 * C)

    obj, reg = pl.pallas_call(
        _head_body,
        grid=(HW_PAD // BN,),
        in_specs=[
            pl.BlockSpec((9 * C, BN), lambda j: (0, j)),
            pl.BlockSpec((C, 9 * C), lambda j: (0, 0)),
            pl.BlockSpec((C, 1), lambda j: (0, 0)),
            pl.BlockSpec((A, C), lambda j: (0, 0)),
            pl.BlockSpec((A, 1), lambda j: (0, 0)),
            pl.BlockSpec((A * 4, C), lambda j: (0, 0)),
            pl.BlockSpec((A * 4, 1), lambda j: (0, 0)),
        ],
        out_specs=[
            pl.BlockSpec((A, BN), lambda j: (0, j)),
            pl.BlockSpec((A * 4, BN), lambda j: (0, j)),
        ],
        out_shape=[
            jax.ShapeDtypeStruct((A, HW_PAD), jnp.float32),
            jax.ShapeDtypeStruct((A * 4, HW_PAD), jnp.float32),
        ],
    )(X, W2, b_conv[:, None], W_cls, b_cls[:, None], W_reg, b_reg[:, None])

    obj_flat = obj.reshape(A, H, WP)[:, :, :W].transpose(1, 2, 0).reshape(-1)
    deltas_flat = reg.reshape(A * 4, H, WP)[:, :, :W].transpose(1, 2, 0).reshape(HW * A, 4)

    top_logits, idx = jax.lax.top_k(obj_flat, PRE_NMS)
    anchors = jnp.asarray(_ANCHORS)
    a_sel = jnp.take(anchors, idx, axis=0).T
    d_sel = jnp.take(deltas_flat, idx, axis=0).T

    boxes_t, scores_m = pl.pallas_call(
        _decode_body,
        out_shape=[
            jax.ShapeDtypeStruct((4, PRE_NMS), jnp.float32),
            jax.ShapeDtypeStruct((1, PRE_NMS), jnp.float32),
        ],
    )(top_logits[None, :], d_sel, a_sel, )

    scores_m = scores_m[0]
    order = jnp.argsort(-scores_m)
    boxes_o = jnp.take(boxes_t.T, order, axis=0)
    scores_o = jnp.take(scores_m, order)

    bx = jnp.pad(boxes_o.T, ((0, 0), (0, NP - PRE_NMS)))
    bc = jnp.pad(boxes_o, ((0, NP - PRE_NMS), (0, 0)))
    sc = jnp.pad(scores_o, (0, NP - PRE_NMS), constant_values=-jnp.inf)[None, :]

    masked = pl.pallas_call(
        _nms_body,
        out_shape=jax.ShapeDtypeStruct((1, NP), jnp.float32),
        scratch_shapes=[pltpu.VMEM((128, 128), jnp.float32)],
    )(bx, bc, sc)

    final_scores, fidx = jax.lax.top_k(masked[0, :PRE_NMS], POST_NMS)
    final_boxes = jnp.take(boxes_o, fidx, axis=0)
    return final_boxes, final_scores, obj_flat, deltas_flat
